# Initial kernel scaffold; baseline (speedup 1.0000x reference)
import jax
import jax.numpy as jnp
from jax.experimental import pallas as pl


def kernel(x, h, edge_index, edge_attr, Wm1, bm1, Wm2, bm2, Wn1, bn1, Wn2, bn2, Wc1, bc1, Wc2, bc2, gamma, beta):
    raise NotImplementedError("write your pallas kernel here")


# speedup vs baseline: 3.1135x; 3.1135x over previous
import jax
import jax.numpy as jnp
from jax import lax
from jax.experimental import pallas as pl
from jax.experimental.pallas import tpu as pltpu
from jax.experimental.pallas import tpu_sc as plsc

N = 10000
E = 320000
H = 128
ED = 16

NPAD = 10240
NC = 2
NS = 16
NW = NC * NS
EPW = E // NW
CH = 80
NCHUNK = EPW // CH
RPT = NPAD // NS


def _silu(v):
    return v * jax.nn.sigmoid(v)


def _node_proj_body(hp, w_hs, w_hd, bm1, ts, td):
    ts[...] = jnp.dot(hp[...], w_hs[...], preferred_element_type=jnp.float32)
    td[...] = jnp.dot(hp[...], w_hd[...], preferred_element_type=jnp.float32) + bm1[...]


def _node_proj(hp, w_hs, w_hd, bm1):
    blk = 512
    return pl.pallas_call(
        _node_proj_body,
        grid=(NPAD // blk,),
        in_specs=[
            pl.BlockSpec((blk, H), lambda i: (i, 0)),
            pl.BlockSpec((H, H), lambda i: (0, 0)),
            pl.BlockSpec((H, H), lambda i: (0, 0)),
            pl.BlockSpec((1, H), lambda i: (0, 0)),
        ],
        out_specs=[
            pl.BlockSpec((blk, H), lambda i: (i, 0)),
            pl.BlockSpec((blk, H), lambda i: (i, 0)),
        ],
        out_shape=[
            jax.ShapeDtypeStruct((NPAD, H), jnp.float32),
            jax.ShapeDtypeStruct((NPAD, H), jnp.float32),
        ],
    )(hp, w_hs, w_hd, bm1)


def _gather_body(tsrc, tdst, src_h, dst_h, x4_h, gs_out, gd_out, d2_out,
                 r0_out, r1_out, r2_out,
                 idx_s, idx_d, x4, rows_s, rows_d, d2buf, rbuf, sem_s, sem_d):
    cid = lax.axis_index("c")
    sid = lax.axis_index("s")
    wid = sid * NC + cid
    base = pl.multiple_of(wid * EPW, 8)
    pltpu.sync_copy(x4_h, x4)
    pltpu.sync_copy(src_h.at[pl.ds(base, EPW)], idx_s)
    pltpu.sync_copy(dst_h.at[pl.ds(base, EPW)], idx_d)

    def chunk(j, carry):
        off = pl.multiple_of(j * CH, 8)
        cs = pltpu.async_copy(tsrc.at[idx_s.at[pl.ds(off, CH)]], rows_s, sem_s)
        cd = pltpu.async_copy(tdst.at[idx_d.at[pl.ds(off, CH)]], rows_d, sem_d)

        def grp(g, c2):
            goff = pl.multiple_of(off + g * 16, 16)
            sv = idx_s[pl.ds(goff, 16)]
            dv = idx_d[pl.ds(goff, 16)]
            acc = jnp.zeros((16,), jnp.float32)
            for comp in range(3):
                xs = plsc.load_gather(x4, [sv * 4 + comp])
                xd = plsc.load_gather(x4, [dv * 4 + comp])
                d = xs - xd
                rbuf[comp, pl.ds(pl.multiple_of(g * 16, 16), 16)] = d
                acc = acc + d * d
            d2buf[pl.ds(pl.multiple_of(g * 16, 16), 16)] = acc
            return c2

        lax.fori_loop(0, CH // 16, grp, 0)
        cs.wait()
        cd.wait()
        pltpu.sync_copy(rows_s, gs_out.at[pl.ds(base + off, CH)])
        pltpu.sync_copy(rows_d, gd_out.at[pl.ds(base + off, CH)])
        pltpu.sync_copy(d2buf, d2_out.at[pl.ds(base + off, CH)])
        pltpu.sync_copy(rbuf.at[0], r0_out.at[pl.ds(base + off, CH)])
        pltpu.sync_copy(rbuf.at[1], r1_out.at[pl.ds(base + off, CH)])
        pltpu.sync_copy(rbuf.at[2], r2_out.at[pl.ds(base + off, CH)])
        return carry

    lax.fori_loop(0, NCHUNK, chunk, 0)


def _gather(tsrc, tdst, src_i, dst_i, x4flat):
    mesh = plsc.VectorSubcoreMesh(core_axis_name="c", subcore_axis_name="s")
    f = pl.kernel(
        _gather_body,
        out_type=[
            jax.ShapeDtypeStruct((E, H), jnp.float32),
            jax.ShapeDtypeStruct((E, H), jnp.float32),
            jax.ShapeDtypeStruct((E,), jnp.float32),
            jax.ShapeDtypeStruct((E,), jnp.float32),
            jax.ShapeDtypeStruct((E,), jnp.float32),
            jax.ShapeDtypeStruct((E,), jnp.float32),
        ],
        mesh=mesh,
        compiler_params=pltpu.CompilerParams(needs_layout_passes=False),
        scratch_types=[
            pltpu.VMEM((EPW,), jnp.int32),
            pltpu.VMEM((EPW,), jnp.int32),
            pltpu.VMEM((NPAD * 4,), jnp.float32),
            pltpu.VMEM((CH, H), jnp.float32),
            pltpu.VMEM((CH, H), jnp.float32),
            pltpu.VMEM((CH,), jnp.float32),
            pltpu.VMEM((3, CH), jnp.float32),
            pltpu.SemaphoreType.DMA,
            pltpu.SemaphoreType.DMA,
        ],
    )
    return f(tsrc, tdst, src_i, dst_i, x4flat)


def _edge_body(gs, gd, ea, d2r, w_ea, w_d2, wm2, bm2, wc1, bc1, wc2, bc2,
               msg_out, w_out):
    d2 = jnp.transpose(d2r[0])
    pre = gs[...] + gd[...] \
        + jnp.dot(ea[...], w_ea[...], preferred_element_type=jnp.float32) \
        + d2 * w_d2[...]
    a1 = _silu(pre)
    msg = jnp.dot(a1, wm2[...], preferred_element_type=jnp.float32) + bm2[...]
    c1 = _silu(jnp.dot(msg, wc1[...], preferred_element_type=jnp.float32) + bc1[...])
    w = jnp.dot(c1, wc2[...], preferred_element_type=jnp.float32) + bc2[...]
    w = jnp.clip(w, -1.0, 1.0)
    msg_out[...] = msg
    w_out[0] = jnp.transpose(w)


def _edge_mlp(gs, gd, ea, d2, w_ea, w_d2, wm2, bm2, wc1, bc1, wc2, bc2):
    blk = 512
    grid = E // blk
    full = lambda shape: pl.BlockSpec(shape, lambda i: tuple(0 for _ in shape))
    return pl.pallas_call(
        _edge_body,
        grid=(grid,),
        in_specs=[
            pl.BlockSpec((blk, H), lambda i: (i, 0)),
            pl.BlockSpec((blk, H), lambda i: (i, 0)),
            pl.BlockSpec((blk, ED), lambda i: (i, 0)),
            pl.BlockSpec((1, 1, blk), lambda i: (i, 0, 0)),
            full((ED, H)),
            full((1, H)),
            full((H, H)),
            full((1, H)),
            full((H, H)),
            full((1, H)),
            full((H, 1)),
            full((1, 1)),
        ],
        out_specs=[
            pl.BlockSpec((blk, H), lambda i: (i, 0)),
            pl.BlockSpec((1, 1, blk), lambda i: (i, 0, 0)),
        ],
        out_shape=[
            jax.ShapeDtypeStruct((E, H), jnp.float32),
            jax.ShapeDtypeStruct((grid, 1, blk), jnp.float32),
        ],
    )(gs, gd, ea, d2.reshape(grid, 1, blk), w_ea, w_d2, wm2, bm2, wc1, bc1,
      wc2, bc2)


def _scatter_body(msg_h, w_h, r0_h, r1_h, r2_h, dst_h, zf_h, z4_h,
                  outf_h, out4_h,
                  accf, acc4, dstc, wbuf, rc, mbuf, updc, idxc, obuf, o4buf):
    cid = lax.axis_index("c")
    sid = lax.axis_index("s")
    wid = sid * NC + cid
    base = pl.multiple_of(wid * EPW, 8)

    zrow = pl.multiple_of(sid * RPT, 8)
    pltpu.sync_copy(zf_h.at[pl.ds(zrow, RPT)], accf.at[pl.ds(zrow, RPT)])
    z4 = pl.multiple_of(sid * (NPAD * 4 // NS), 8)
    pltpu.sync_copy(z4_h.at[pl.ds(z4, NPAD * 4 // NS)],
                    acc4.at[pl.ds(z4, NPAD * 4 // NS)])
    plsc.subcore_barrier()

    def chunk(j, carry):
        off = pl.multiple_of(j * CH, 8)
        pltpu.sync_copy(dst_h.at[pl.ds(base + off, CH)], dstc)
        pltpu.sync_copy(w_h.at[pl.ds(base + off, CH)], wbuf)
        pltpu.sync_copy(r0_h.at[pl.ds(base + off, CH)], rc.at[0])
        pltpu.sync_copy(r1_h.at[pl.ds(base + off, CH)], rc.at[1])
        pltpu.sync_copy(r2_h.at[pl.ds(base + off, CH)], rc.at[2])
        pltpu.sync_copy(msg_h.at[pl.ds(base + off, CH)], mbuf)

        def grp(g, c2):
            goff = pl.multiple_of(g * 16, 16)
            dv = dstc[pl.ds(goff, 16)]
            wv = wbuf[pl.ds(goff, 16)]
            for comp in range(3):
                rv = rc[comp, pl.ds(goff, 16)]
                updc[comp, pl.ds(goff, 16)] = wv * rv
                idxc[comp, pl.ds(goff, 16)] = dv * 4 + comp
            return c2

        lax.fori_loop(0, CH // 16, grp, 0)
        pltpu.sync_copy(mbuf, accf.at[dstc], add=True)
        for comp in range(3):
            pltpu.sync_copy(updc.at[comp], acc4.at[idxc.at[comp]], add=True)
        return carry

    lax.fori_loop(0, NCHUNK, chunk, 0)
    plsc.subcore_barrier()

    for k in range(8):
        row0 = pl.multiple_of(sid * RPT + k * (RPT // 8), 8)
        pltpu.sync_copy(accf.at[pl.ds(row0, RPT // 8)], obuf)
        pltpu.sync_copy(obuf, outf_h.at[cid].at[pl.ds(row0, RPT // 8)])
    pltpu.sync_copy(acc4.at[pl.ds(z4, NPAD * 4 // NS)], o4buf)
    pltpu.sync_copy(o4buf, out4_h.at[cid].at[pl.ds(z4, NPAD * 4 // NS)])


def _scatter(msg, w1d, r0, r1, r2, dst_i, zerosf, zeros4):
    mesh = plsc.VectorSubcoreMesh(core_axis_name="c", subcore_axis_name="s")
    f = pl.kernel(
        _scatter_body,
        out_type=[
            jax.ShapeDtypeStruct((NC, NPAD, H), jnp.float32),
            jax.ShapeDtypeStruct((NC, NPAD * 4), jnp.float32),
        ],
        mesh=mesh,
        compiler_params=pltpu.CompilerParams(needs_layout_passes=False),
        scratch_types=[
            pltpu.VMEM_SHARED((NPAD, H), jnp.float32),
            pltpu.VMEM_SHARED((NPAD * 4,), jnp.float32),
            pltpu.VMEM((CH,), jnp.int32),
            pltpu.VMEM((CH,), jnp.float32),
            pltpu.VMEM((3, CH), jnp.float32),
            pltpu.VMEM((CH, H), jnp.float32),
            pltpu.VMEM((3, CH), jnp.float32),
            pltpu.VMEM((3, CH), jnp.int32),
            pltpu.VMEM((RPT // 8, H), jnp.float32),
            pltpu.VMEM((NPAD * 4 // NS,), jnp.float32),
        ],
    )
    return f(msg, w1d, r0, r1, r2, dst_i, zerosf, zeros4)


def _update_body(hp, pf0, pf1, p40, p41, xp4, wn1h, wn1a, bn1, wn2, bn2,
                 gamma, beta, hout, xout):
    agg = pf0[...] + pf1[...]
    u = _silu(jnp.dot(hp[...], wn1h[...], preferred_element_type=jnp.float32)
              + jnp.dot(agg, wn1a[...], preferred_element_type=jnp.float32)
              + bn1[...])
    hn = hp[...] + jnp.dot(u, wn2[...], preferred_element_type=jnp.float32) + bn2[...]
    mean = jnp.mean(hn, axis=1, keepdims=True)
    zc = hn - mean
    var = jnp.mean(zc * zc, axis=1, keepdims=True)
    hout[...] = zc * lax.rsqrt(var + 1e-5) * gamma[...] + beta[...]
    xout[...] = xp4[...] + p40[...] + p41[...]


def _node_update(hp, pf0, pf1, p40, p41, xp4, wn1h, wn1a, bn1, wn2, bn2,
                 gamma, beta):
    blk = 512
    grid = NPAD // blk
    full = lambda shape: pl.BlockSpec(shape, lambda i: tuple(0 for _ in shape))
    return pl.pallas_call(
        _update_body,
        grid=(grid,),
        in_specs=[
            pl.BlockSpec((blk, H), lambda i: (i, 0)),
            pl.BlockSpec((blk, H), lambda i: (i, 0)),
            pl.BlockSpec((blk, H), lambda i: (i, 0)),
            pl.BlockSpec((blk * 4,), lambda i: (i,)),
            pl.BlockSpec((blk * 4,), lambda i: (i,)),
            pl.BlockSpec((blk * 4,), lambda i: (i,)),
            full((H, H)),
            full((H, H)),
            full((1, H)),
            full((H, H)),
            full((1, H)),
            full((1, H)),
            full((1, H)),
        ],
        out_specs=[
            pl.BlockSpec((blk, H), lambda i: (i, 0)),
            pl.BlockSpec((blk * 4,), lambda i: (i,)),
        ],
        out_shape=[
            jax.ShapeDtypeStruct((NPAD, H), jnp.float32),
            jax.ShapeDtypeStruct((NPAD * 4,), jnp.float32),
        ],
    )(hp, pf0, pf1, p40, p41, xp4, wn1h, wn1a, bn1, wn2, bn2, gamma, beta)


def kernel(x, h, edge_index, edge_attr, Wm1, bm1, Wm2, bm2, Wn1, bn1,
           Wn2, bn2, Wc1, bc1, Wc2, bc2, gamma, beta):
    src = edge_index[0].astype(jnp.int32)
    dst = edge_index[1].astype(jnp.int32)
    x4 = jnp.zeros((NPAD, 4), jnp.float32).at[:N, :3].set(x)
    x4flat = x4.reshape(NPAD * 4)
    hp = jnp.zeros((NPAD, H), jnp.float32).at[:N].set(h)

    w_hs = Wm1[:H]
    w_hd = Wm1[H:2 * H]
    w_ea = Wm1[2 * H:2 * H + ED]
    w_d2 = Wm1[2 * H + ED:]

    tsrc, tdst = _node_proj(hp, w_hs, w_hd, bm1.reshape(1, H))
    gs, gd, d2, r0, r1, r2 = _gather(tsrc, tdst, src, dst, x4flat)
    msg, w2d = _edge_mlp(gs, gd, edge_attr, d2, w_ea, w_d2, Wm2,
                         bm2.reshape(1, H), Wc1, bc1.reshape(1, H),
                         Wc2.reshape(H, 1), bc2.reshape(1, 1))
    pf, p4 = _scatter(msg, w2d.reshape(E), r0, r1, r2, dst,
                      jnp.zeros((NPAD, H), jnp.float32),
                      jnp.zeros((NPAD * 4,), jnp.float32))
    hn, xn = _node_update(hp, pf[0], pf[1], p4[0], p4[1], x4flat,
                          Wn1[:H], Wn1[H:], bn1.reshape(1, H),
                          Wn2, bn2.reshape(1, H),
                          gamma.reshape(1, H), beta.reshape(1, H))
    return (xn.reshape(NPAD, 4)[:N, :3], hn[:N])


# speedup vs baseline: 3.1137x; 1.0001x over previous
import jax
import jax.numpy as jnp
from jax import lax
from jax.experimental import pallas as pl
from jax.experimental.pallas import tpu as pltpu
from jax.experimental.pallas import tpu_sc as plsc

N = 10000
E = 320000
H = 128
ED = 16

NPAD = 10240
NC = 2
NS = 16
NW = NC * NS
EPW = E // NW
CH = 80
NCHUNK = EPW // CH
RPT = NPAD // NS


def _silu(v):
    return v * jax.nn.sigmoid(v)


def _node_proj_body(hp, w_hs, w_hd, bm1, ts, td):
    ts[...] = jnp.dot(hp[...], w_hs[...], preferred_element_type=jnp.float32)
    td[...] = jnp.dot(hp[...], w_hd[...], preferred_element_type=jnp.float32) + bm1[...]


def _node_proj(hp, w_hs, w_hd, bm1):
    blk = 512
    return pl.pallas_call(
        _node_proj_body,
        grid=(NPAD // blk,),
        in_specs=[
            pl.BlockSpec((blk, H), lambda i: (i, 0)),
            pl.BlockSpec((H, H), lambda i: (0, 0)),
            pl.BlockSpec((H, H), lambda i: (0, 0)),
            pl.BlockSpec((1, H), lambda i: (0, 0)),
        ],
        out_specs=[
            pl.BlockSpec((blk, H), lambda i: (i, 0)),
            pl.BlockSpec((blk, H), lambda i: (i, 0)),
        ],
        out_shape=[
            jax.ShapeDtypeStruct((NPAD, H), jnp.float32),
            jax.ShapeDtypeStruct((NPAD, H), jnp.float32),
        ],
    )(hp, w_hs, w_hd, bm1)


def _gather_body(tsrc, tdst, src_h, dst_h, x4_h, gs_out, gd_out, d2_out,
                 r0_out, r1_out, r2_out,
                 idx_s, idx_d, x4, rows_s, rows_d, d2buf, rbuf, sem_s, sem_d):
    cid = lax.axis_index("c")
    sid = lax.axis_index("s")
    wid = sid * NC + cid
    base = pl.multiple_of(wid * EPW, 8)
    pltpu.sync_copy(x4_h, x4)
    pltpu.sync_copy(src_h.at[pl.ds(base, EPW)], idx_s)
    pltpu.sync_copy(dst_h.at[pl.ds(base, EPW)], idx_d)

    def chunk(j, carry):
        off = pl.multiple_of(j * CH, 8)
        cs = pltpu.async_copy(tsrc.at[idx_s.at[pl.ds(off, CH)]], rows_s, sem_s)
        cd = pltpu.async_copy(tdst.at[idx_d.at[pl.ds(off, CH)]], rows_d, sem_d)

        def grp(g, c2):
            goff = pl.multiple_of(off + g * 16, 16)
            sv = idx_s[pl.ds(goff, 16)]
            dv = idx_d[pl.ds(goff, 16)]
            acc = jnp.zeros((16,), jnp.float32)
            for comp in range(3):
                xs = plsc.load_gather(x4, [sv * 4 + comp])
                xd = plsc.load_gather(x4, [dv * 4 + comp])
                d = xs - xd
                rbuf[comp, pl.ds(pl.multiple_of(g * 16, 16), 16)] = d
                acc = acc + d * d
            d2buf[pl.ds(pl.multiple_of(g * 16, 16), 16)] = acc
            return c2

        lax.fori_loop(0, CH // 16, grp, 0)
        cs.wait()
        cd.wait()
        pltpu.sync_copy(rows_s, gs_out.at[pl.ds(base + off, CH)])
        pltpu.sync_copy(rows_d, gd_out.at[pl.ds(base + off, CH)])
        pltpu.sync_copy(d2buf, d2_out.at[pl.ds(base + off, CH)])
        pltpu.sync_copy(rbuf.at[0], r0_out.at[pl.ds(base + off, CH)])
        pltpu.sync_copy(rbuf.at[1], r1_out.at[pl.ds(base + off, CH)])
        pltpu.sync_copy(rbuf.at[2], r2_out.at[pl.ds(base + off, CH)])
        return carry

    lax.fori_loop(0, NCHUNK, chunk, 0)


def _gather(tsrc, tdst, src_i, dst_i, x4flat):
    mesh = plsc.VectorSubcoreMesh(core_axis_name="c", subcore_axis_name="s")
    f = pl.kernel(
        _gather_body,
        out_type=[
            jax.ShapeDtypeStruct((E, H), jnp.float32),
            jax.ShapeDtypeStruct((E, H), jnp.float32),
            jax.ShapeDtypeStruct((E,), jnp.float32),
            jax.ShapeDtypeStruct((E,), jnp.float32),
            jax.ShapeDtypeStruct((E,), jnp.float32),
            jax.ShapeDtypeStruct((E,), jnp.float32),
        ],
        mesh=mesh,
        compiler_params=pltpu.CompilerParams(needs_layout_passes=False),
        scratch_types=[
            pltpu.VMEM((EPW,), jnp.int32),
            pltpu.VMEM((EPW,), jnp.int32),
            pltpu.VMEM((NPAD * 4,), jnp.float32),
            pltpu.VMEM((CH, H), jnp.float32),
            pltpu.VMEM((CH, H), jnp.float32),
            pltpu.VMEM((CH,), jnp.float32),
            pltpu.VMEM((3, CH), jnp.float32),
            pltpu.SemaphoreType.DMA,
            pltpu.SemaphoreType.DMA,
        ],
    )
    return f(tsrc, tdst, src_i, dst_i, x4flat)


def _edge_body(gs, gd, ea, d2r, w_ea, w_d2, wm2, bm2, wc1, bc1, wc2, bc2,
               msg_out, w_out):
    d2 = jnp.transpose(d2r[0])
    pre = gs[...] + gd[...] \
        + jnp.dot(ea[...], w_ea[...], preferred_element_type=jnp.float32) \
        + d2 * w_d2[...]
    a1 = _silu(pre).astype(jnp.bfloat16)
    msg = jnp.dot(a1, wm2[...].astype(jnp.bfloat16),
                  preferred_element_type=jnp.float32) + bm2[...]
    c1 = _silu(jnp.dot(msg.astype(jnp.bfloat16), wc1[...].astype(jnp.bfloat16),
                       preferred_element_type=jnp.float32) + bc1[...])
    w = jnp.dot(c1.astype(jnp.bfloat16), wc2[...].astype(jnp.bfloat16),
                preferred_element_type=jnp.float32) + bc2[...]
    w = jnp.clip(w, -1.0, 1.0)
    msg_out[...] = msg
    w_out[0] = jnp.transpose(w)


def _edge_mlp(gs, gd, ea, d2, w_ea, w_d2, wm2, bm2, wc1, bc1, wc2, bc2):
    blk = 512
    grid = E // blk
    full = lambda shape: pl.BlockSpec(shape, lambda i: tuple(0 for _ in shape))
    return pl.pallas_call(
        _edge_body,
        grid=(grid,),
        in_specs=[
            pl.BlockSpec((blk, H), lambda i: (i, 0)),
            pl.BlockSpec((blk, H), lambda i: (i, 0)),
            pl.BlockSpec((blk, ED), lambda i: (i, 0)),
            pl.BlockSpec((1, 1, blk), lambda i: (i, 0, 0)),
            full((ED, H)),
            full((1, H)),
            full((H, H)),
            full((1, H)),
            full((H, H)),
            full((1, H)),
            full((H, 1)),
            full((1, 1)),
        ],
        out_specs=[
            pl.BlockSpec((blk, H), lambda i: (i, 0)),
            pl.BlockSpec((1, 1, blk), lambda i: (i, 0, 0)),
        ],
        out_shape=[
            jax.ShapeDtypeStruct((E, H), jnp.float32),
            jax.ShapeDtypeStruct((grid, 1, blk), jnp.float32),
        ],
    )(gs, gd, ea, d2.reshape(grid, 1, blk), w_ea, w_d2, wm2, bm2, wc1, bc1,
      wc2, bc2)


def _scatter_body(msg_h, w_h, r0_h, r1_h, r2_h, dst_h, zf_h, z4_h,
                  outf_h, out4_h,
                  accf, acc4, dstc, wbuf, rc, mbuf, updc, idxc, obuf, o4buf):
    cid = lax.axis_index("c")
    sid = lax.axis_index("s")
    wid = sid * NC + cid
    base = pl.multiple_of(wid * EPW, 8)

    zrow = pl.multiple_of(sid * RPT, 8)
    pltpu.sync_copy(zf_h.at[pl.ds(zrow, RPT)], accf.at[pl.ds(zrow, RPT)])
    z4 = pl.multiple_of(sid * (NPAD * 4 // NS), 8)
    pltpu.sync_copy(z4_h.at[pl.ds(z4, NPAD * 4 // NS)],
                    acc4.at[pl.ds(z4, NPAD * 4 // NS)])
    plsc.subcore_barrier()

    def chunk(j, carry):
        off = pl.multiple_of(j * CH, 8)
        pltpu.sync_copy(dst_h.at[pl.ds(base + off, CH)], dstc)
        pltpu.sync_copy(w_h.at[pl.ds(base + off, CH)], wbuf)
        pltpu.sync_copy(r0_h.at[pl.ds(base + off, CH)], rc.at[0])
        pltpu.sync_copy(r1_h.at[pl.ds(base + off, CH)], rc.at[1])
        pltpu.sync_copy(r2_h.at[pl.ds(base + off, CH)], rc.at[2])
        pltpu.sync_copy(msg_h.at[pl.ds(base + off, CH)], mbuf)

        def grp(g, c2):
            goff = pl.multiple_of(g * 16, 16)
            dv = dstc[pl.ds(goff, 16)]
            wv = wbuf[pl.ds(goff, 16)]
            for comp in range(3):
                rv = rc[comp, pl.ds(goff, 16)]
                updc[comp, pl.ds(goff, 16)] = wv * rv
                idxc[comp, pl.ds(goff, 16)] = dv * 4 + comp
            return c2

        lax.fori_loop(0, CH // 16, grp, 0)
        pltpu.sync_copy(mbuf, accf.at[dstc], add=True)
        for comp in range(3):
            pltpu.sync_copy(updc.at[comp], acc4.at[idxc.at[comp]], add=True)
        return carry

    lax.fori_loop(0, NCHUNK, chunk, 0)
    plsc.subcore_barrier()

    for k in range(8):
        row0 = pl.multiple_of(sid * RPT + k * (RPT // 8), 8)
        pltpu.sync_copy(accf.at[pl.ds(row0, RPT // 8)], obuf)
        pltpu.sync_copy(obuf, outf_h.at[cid].at[pl.ds(row0, RPT // 8)])
    pltpu.sync_copy(acc4.at[pl.ds(z4, NPAD * 4 // NS)], o4buf)
    pltpu.sync_copy(o4buf, out4_h.at[cid].at[pl.ds(z4, NPAD * 4 // NS)])


def _scatter(msg, w1d, r0, r1, r2, dst_i, zerosf, zeros4):
    mesh = plsc.VectorSubcoreMesh(core_axis_name="c", subcore_axis_name="s")
    f = pl.kernel(
        _scatter_body,
        out_type=[
            jax.ShapeDtypeStruct((NC, NPAD, H), jnp.float32),
            jax.ShapeDtypeStruct((NC, NPAD * 4), jnp.float32),
        ],
        mesh=mesh,
        compiler_params=pltpu.CompilerParams(needs_layout_passes=False),
        scratch_types=[
            pltpu.VMEM_SHARED((NPAD, H), jnp.float32),
            pltpu.VMEM_SHARED((NPAD * 4,), jnp.float32),
            pltpu.VMEM((CH,), jnp.int32),
            pltpu.VMEM((CH,), jnp.float32),
            pltpu.VMEM((3, CH), jnp.float32),
            pltpu.VMEM((CH, H), jnp.float32),
            pltpu.VMEM((3, CH), jnp.float32),
            pltpu.VMEM((3, CH), jnp.int32),
            pltpu.VMEM((RPT // 8, H), jnp.float32),
            pltpu.VMEM((NPAD * 4 // NS,), jnp.float32),
        ],
    )
    return f(msg, w1d, r0, r1, r2, dst_i, zerosf, zeros4)


def _update_body(hp, pf0, pf1, p40, p41, xp4, wn1h, wn1a, bn1, wn2, bn2,
                 gamma, beta, hout, xout):
    agg = pf0[...] + pf1[...]
    u = _silu(jnp.dot(hp[...], wn1h[...], preferred_element_type=jnp.float32)
              + jnp.dot(agg, wn1a[...], preferred_element_type=jnp.float32)
              + bn1[...])
    hn = hp[...] + jnp.dot(u, wn2[...], preferred_element_type=jnp.float32) + bn2[...]
    mean = jnp.mean(hn, axis=1, keepdims=True)
    zc = hn - mean
    var = jnp.mean(zc * zc, axis=1, keepdims=True)
    hout[...] = zc * lax.rsqrt(var + 1e-5) * gamma[...] + beta[...]
    xout[...] = xp4[...] + p40[...] + p41[...]


def _node_update(hp, pf0, pf1, p40, p41, xp4, wn1h, wn1a, bn1, wn2, bn2,
                 gamma, beta):
    blk = 512
    grid = NPAD // blk
    full = lambda shape: pl.BlockSpec(shape, lambda i: tuple(0 for _ in shape))
    return pl.pallas_call(
        _update_body,
        grid=(grid,),
        in_specs=[
            pl.BlockSpec((blk, H), lambda i: (i, 0)),
            pl.BlockSpec((blk, H), lambda i: (i, 0)),
            pl.BlockSpec((blk, H), lambda i: (i, 0)),
            pl.BlockSpec((blk * 4,), lambda i: (i,)),
            pl.BlockSpec((blk * 4,), lambda i: (i,)),
            pl.BlockSpec((blk * 4,), lambda i: (i,)),
            full((H, H)),
            full((H, H)),
            full((1, H)),
            full((H, H)),
            full((1, H)),
            full((1, H)),
            full((1, H)),
        ],
        out_specs=[
            pl.BlockSpec((blk, H), lambda i: (i, 0)),
            pl.BlockSpec((blk * 4,), lambda i: (i,)),
        ],
        out_shape=[
            jax.ShapeDtypeStruct((NPAD, H), jnp.float32),
            jax.ShapeDtypeStruct((NPAD * 4,), jnp.float32),
        ],
    )(hp, pf0, pf1, p40, p41, xp4, wn1h, wn1a, bn1, wn2, bn2, gamma, beta)


def kernel(x, h, edge_index, edge_attr, Wm1, bm1, Wm2, bm2, Wn1, bn1,
           Wn2, bn2, Wc1, bc1, Wc2, bc2, gamma, beta):
    src = edge_index[0].astype(jnp.int32)
    dst = edge_index[1].astype(jnp.int32)
    x4 = jnp.zeros((NPAD, 4), jnp.float32).at[:N, :3].set(x)
    x4flat = x4.reshape(NPAD * 4)
    hp = jnp.zeros((NPAD, H), jnp.float32).at[:N].set(h)

    w_hs = Wm1[:H]
    w_hd = Wm1[H:2 * H]
    w_ea = Wm1[2 * H:2 * H + ED]
    w_d2 = Wm1[2 * H + ED:]

    tsrc, tdst = _node_proj(hp, w_hs, w_hd, bm1.reshape(1, H))
    gs, gd, d2, r0, r1, r2 = _gather(tsrc, tdst, src, dst, x4flat)
    msg, w2d = _edge_mlp(gs, gd, edge_attr, d2, w_ea, w_d2, Wm2,
                         bm2.reshape(1, H), Wc1, bc1.reshape(1, H),
                         Wc2.reshape(H, 1), bc2.reshape(1, 1))
    pf, p4 = _scatter(msg, w2d.reshape(E), r0, r1, r2, dst,
                      jnp.zeros((NPAD, H), jnp.float32),
                      jnp.zeros((NPAD * 4,), jnp.float32))
    hn, xn = _node_update(hp, pf[0], pf[1], p4[0], p4[1], x4flat,
                          Wn1[:H], Wn1[H:], bn1.reshape(1, H),
                          Wn2, bn2.reshape(1, H),
                          gamma.reshape(1, H), beta.reshape(1, H))
    return (xn.reshape(NPAD, 4)[:N, :3], hn[:N])


# speedup vs baseline: 3.7848x; 1.2155x over previous
import jax
import jax.numpy as jnp
from jax import lax
from jax.experimental import pallas as pl
from jax.experimental.pallas import tpu as pltpu
from jax.experimental.pallas import tpu_sc as plsc

N = 10000
E = 320000
H = 128
ED = 16

NPAD = 10240
NC = 2
NS = 16
NW = NC * NS
EPW = E // NW
CH = 80
NCHUNK = EPW // CH
RPT = NPAD // NS


def _silu(v):
    return v * jax.nn.sigmoid(v)


def _node_proj_body(hp, w_hs, w_hd, bm1, ts, td):
    ts[...] = jnp.dot(hp[...], w_hs[...], preferred_element_type=jnp.float32)
    td[...] = jnp.dot(hp[...], w_hd[...], preferred_element_type=jnp.float32) + bm1[...]


def _node_proj(hp, w_hs, w_hd, bm1):
    blk = 512
    return pl.pallas_call(
        _node_proj_body,
        grid=(NPAD // blk,),
        in_specs=[
            pl.BlockSpec((blk, H), lambda i: (i, 0)),
            pl.BlockSpec((H, H), lambda i: (0, 0)),
            pl.BlockSpec((H, H), lambda i: (0, 0)),
            pl.BlockSpec((1, H), lambda i: (0, 0)),
        ],
        out_specs=[
            pl.BlockSpec((blk, H), lambda i: (i, 0)),
            pl.BlockSpec((blk, H), lambda i: (i, 0)),
        ],
        out_shape=[
            jax.ShapeDtypeStruct((NPAD, H), jnp.float32),
            jax.ShapeDtypeStruct((NPAD, H), jnp.float32),
        ],
    )(hp, w_hs, w_hd, bm1)


def _gather_body(tsrc, tdst, src_h, dst_h, x4_h, gs_out, gd_out, d2_out,
                 r0_out, r1_out, r2_out,
                 idx_s, idx_d, x4, rows_s, rows_d, d2buf, rbuf, sem_s, sem_d):
    cid = lax.axis_index("c")
    sid = lax.axis_index("s")
    wid = sid * NC + cid
    base = pl.multiple_of(wid * EPW, 8)
    pltpu.sync_copy(x4_h, x4)
    pltpu.sync_copy(src_h.at[pl.ds(base, EPW)], idx_s)
    pltpu.sync_copy(dst_h.at[pl.ds(base, EPW)], idx_d)

    def chunk(j, carry):
        off = pl.multiple_of(j * CH, 8)
        cs = pltpu.async_copy(tsrc.at[idx_s.at[pl.ds(off, CH)]], rows_s, sem_s)
        cd = pltpu.async_copy(tdst.at[idx_d.at[pl.ds(off, CH)]], rows_d, sem_d)

        def grp(g, c2):
            goff = pl.multiple_of(off + g * 16, 16)
            sv = idx_s[pl.ds(goff, 16)]
            dv = idx_d[pl.ds(goff, 16)]
            acc = jnp.zeros((16,), jnp.float32)
            for comp in range(3):
                xs = plsc.load_gather(x4, [sv * 4 + comp])
                xd = plsc.load_gather(x4, [dv * 4 + comp])
                d = xs - xd
                rbuf[comp, pl.ds(pl.multiple_of(g * 16, 16), 16)] = d
                acc = acc + d * d
            d2buf[pl.ds(pl.multiple_of(g * 16, 16), 16)] = acc
            return c2

        lax.fori_loop(0, CH // 16, grp, 0)
        cs.wait()
        cd.wait()
        pltpu.sync_copy(rows_s, gs_out.at[pl.ds(base + off, CH)])
        pltpu.sync_copy(rows_d, gd_out.at[pl.ds(base + off, CH)])
        pltpu.sync_copy(d2buf, d2_out.at[pl.ds(base + off, CH)])
        pltpu.sync_copy(rbuf.at[0], r0_out.at[pl.ds(base + off, CH)])
        pltpu.sync_copy(rbuf.at[1], r1_out.at[pl.ds(base + off, CH)])
        pltpu.sync_copy(rbuf.at[2], r2_out.at[pl.ds(base + off, CH)])
        return carry

    lax.fori_loop(0, NCHUNK, chunk, 0)


def _gather(tsrc, tdst, src_i, dst_i, x4flat):
    mesh = plsc.VectorSubcoreMesh(core_axis_name="c", subcore_axis_name="s")
    f = pl.kernel(
        _gather_body,
        out_type=[
            jax.ShapeDtypeStruct((E, H), jnp.float32),
            jax.ShapeDtypeStruct((E, H), jnp.float32),
            jax.ShapeDtypeStruct((E,), jnp.float32),
            jax.ShapeDtypeStruct((E,), jnp.float32),
            jax.ShapeDtypeStruct((E,), jnp.float32),
            jax.ShapeDtypeStruct((E,), jnp.float32),
        ],
        mesh=mesh,
        compiler_params=pltpu.CompilerParams(needs_layout_passes=False),
        scratch_types=[
            pltpu.VMEM((EPW,), jnp.int32),
            pltpu.VMEM((EPW,), jnp.int32),
            pltpu.VMEM((NPAD * 4,), jnp.float32),
            pltpu.VMEM((CH, H), jnp.float32),
            pltpu.VMEM((CH, H), jnp.float32),
            pltpu.VMEM((CH,), jnp.float32),
            pltpu.VMEM((3, CH), jnp.float32),
            pltpu.SemaphoreType.DMA,
            pltpu.SemaphoreType.DMA,
        ],
    )
    return f(tsrc, tdst, src_i, dst_i, x4flat)


def _edge_body(gs, gd, ea, d2r, w_ea, w_d2, wm2, bm2, wc1, bc1, wc2, bc2,
               msg_out, w_out):
    d2 = jnp.transpose(d2r[0])
    pre = gs[...] + gd[...] \
        + jnp.dot(ea[...], w_ea[...], preferred_element_type=jnp.float32) \
        + d2 * w_d2[...]
    a1 = _silu(pre).astype(jnp.bfloat16)
    msg = jnp.dot(a1, wm2[...].astype(jnp.bfloat16),
                  preferred_element_type=jnp.float32) + bm2[...]
    c1 = _silu(jnp.dot(msg.astype(jnp.bfloat16), wc1[...].astype(jnp.bfloat16),
                       preferred_element_type=jnp.float32) + bc1[...])
    w = jnp.dot(c1.astype(jnp.bfloat16), wc2[...].astype(jnp.bfloat16),
                preferred_element_type=jnp.float32) + bc2[...]
    w = jnp.clip(w, -1.0, 1.0)
    msg_out[...] = msg
    w_out[0] = jnp.transpose(w)


def _edge_mlp(gs, gd, ea, d2, w_ea, w_d2, wm2, bm2, wc1, bc1, wc2, bc2):
    blk = 1280
    grid = E // blk
    full = lambda shape: pl.BlockSpec(shape, lambda i: tuple(0 for _ in shape))
    return pl.pallas_call(
        _edge_body,
        grid=(grid,),
        in_specs=[
            pl.BlockSpec((blk, H), lambda i: (i, 0)),
            pl.BlockSpec((blk, H), lambda i: (i, 0)),
            pl.BlockSpec((blk, ED), lambda i: (i, 0)),
            pl.BlockSpec((1, 1, blk), lambda i: (i, 0, 0)),
            full((ED, H)),
            full((1, H)),
            full((H, H)),
            full((1, H)),
            full((H, H)),
            full((1, H)),
            full((H, 1)),
            full((1, 1)),
        ],
        out_specs=[
            pl.BlockSpec((blk, H), lambda i: (i, 0)),
            pl.BlockSpec((1, 1, blk), lambda i: (i, 0, 0)),
        ],
        out_shape=[
            jax.ShapeDtypeStruct((E, H), jnp.float32),
            jax.ShapeDtypeStruct((grid, 1, blk), jnp.float32),
        ],
    )(gs, gd, ea, d2.reshape(grid, 1, blk), w_ea, w_d2, wm2, bm2, wc1, bc1,
      wc2, bc2)


def _scatter_body(msg_h, w_h, r0_h, r1_h, r2_h, dst_h, zf_h, z4_h,
                  outf_h, out4_h,
                  accf, acc4, dstc, wbuf, rc, mbuf, updc, idxc, obuf, o4buf):
    cid = lax.axis_index("c")
    sid = lax.axis_index("s")
    wid = sid * NC + cid
    base = pl.multiple_of(wid * EPW, 8)

    zrow = pl.multiple_of(sid * RPT, 8)
    pltpu.sync_copy(zf_h.at[pl.ds(zrow, RPT)], accf.at[pl.ds(zrow, RPT)])
    z4 = pl.multiple_of(sid * (NPAD * 4 // NS), 8)
    pltpu.sync_copy(z4_h.at[pl.ds(z4, NPAD * 4 // NS)],
                    acc4.at[pl.ds(z4, NPAD * 4 // NS)])
    plsc.subcore_barrier()

    def chunk(j, carry):
        off = pl.multiple_of(j * CH, 8)
        pltpu.sync_copy(dst_h.at[pl.ds(base + off, CH)], dstc)
        pltpu.sync_copy(w_h.at[pl.ds(base + off, CH)], wbuf)
        pltpu.sync_copy(r0_h.at[pl.ds(base + off, CH)], rc.at[0])
        pltpu.sync_copy(r1_h.at[pl.ds(base + off, CH)], rc.at[1])
        pltpu.sync_copy(r2_h.at[pl.ds(base + off, CH)], rc.at[2])
        pltpu.sync_copy(msg_h.at[pl.ds(base + off, CH)], mbuf)

        def grp(g, c2):
            goff = pl.multiple_of(g * 16, 16)
            dv = dstc[pl.ds(goff, 16)]
            wv = wbuf[pl.ds(goff, 16)]
            for comp in range(3):
                rv = rc[comp, pl.ds(goff, 16)]
                updc[comp, pl.ds(goff, 16)] = wv * rv
                idxc[comp, pl.ds(goff, 16)] = dv * 4 + comp
            return c2

        lax.fori_loop(0, CH // 16, grp, 0)
        pltpu.sync_copy(mbuf, accf.at[dstc], add=True)
        for comp in range(3):
            pltpu.sync_copy(updc.at[comp], acc4.at[idxc.at[comp]], add=True)
        return carry

    lax.fori_loop(0, NCHUNK, chunk, 0)
    plsc.subcore_barrier()

    for k in range(8):
        row0 = pl.multiple_of(sid * RPT + k * (RPT // 8), 8)
        pltpu.sync_copy(accf.at[pl.ds(row0, RPT // 8)], obuf)
        pltpu.sync_copy(obuf, outf_h.at[cid].at[pl.ds(row0, RPT // 8)])
    pltpu.sync_copy(acc4.at[pl.ds(z4, NPAD * 4 // NS)], o4buf)
    pltpu.sync_copy(o4buf, out4_h.at[cid].at[pl.ds(z4, NPAD * 4 // NS)])


def _scatter(msg, w1d, r0, r1, r2, dst_i, zerosf, zeros4):
    mesh = plsc.VectorSubcoreMesh(core_axis_name="c", subcore_axis_name="s")
    f = pl.kernel(
        _scatter_body,
        out_type=[
            jax.ShapeDtypeStruct((NC, NPAD, H), jnp.float32),
            jax.ShapeDtypeStruct((NC, NPAD * 4), jnp.float32),
        ],
        mesh=mesh,
        compiler_params=pltpu.CompilerParams(needs_layout_passes=False),
        scratch_types=[
            pltpu.VMEM_SHARED((NPAD, H), jnp.float32),
            pltpu.VMEM_SHARED((NPAD * 4,), jnp.float32),
            pltpu.VMEM((CH,), jnp.int32),
            pltpu.VMEM((CH,), jnp.float32),
            pltpu.VMEM((3, CH), jnp.float32),
            pltpu.VMEM((CH, H), jnp.float32),
            pltpu.VMEM((3, CH), jnp.float32),
            pltpu.VMEM((3, CH), jnp.int32),
            pltpu.VMEM((RPT // 8, H), jnp.float32),
            pltpu.VMEM((NPAD * 4 // NS,), jnp.float32),
        ],
    )
    return f(msg, w1d, r0, r1, r2, dst_i, zerosf, zeros4)


def _update_body(hp, pf0, pf1, p40, p41, xp4, wn1h, wn1a, bn1, wn2, bn2,
                 gamma, beta, hout, xout):
    agg = pf0[...] + pf1[...]
    u = _silu(jnp.dot(hp[...], wn1h[...], preferred_element_type=jnp.float32)
              + jnp.dot(agg, wn1a[...], preferred_element_type=jnp.float32)
              + bn1[...])
    hn = hp[...] + jnp.dot(u, wn2[...], preferred_element_type=jnp.float32) + bn2[...]
    mean = jnp.mean(hn, axis=1, keepdims=True)
    zc = hn - mean
    var = jnp.mean(zc * zc, axis=1, keepdims=True)
    hout[...] = zc * lax.rsqrt(var + 1e-5) * gamma[...] + beta[...]
    xout[...] = xp4[...] + p40[...] + p41[...]


def _node_update(hp, pf0, pf1, p40, p41, xp4, wn1h, wn1a, bn1, wn2, bn2,
                 gamma, beta):
    blk = 512
    grid = NPAD // blk
    full = lambda shape: pl.BlockSpec(shape, lambda i: tuple(0 for _ in shape))
    return pl.pallas_call(
        _update_body,
        grid=(grid,),
        in_specs=[
            pl.BlockSpec((blk, H), lambda i: (i, 0)),
            pl.BlockSpec((blk, H), lambda i: (i, 0)),
            pl.BlockSpec((blk, H), lambda i: (i, 0)),
            pl.BlockSpec((blk * 4,), lambda i: (i,)),
            pl.BlockSpec((blk * 4,), lambda i: (i,)),
            pl.BlockSpec((blk * 4,), lambda i: (i,)),
            full((H, H)),
            full((H, H)),
            full((1, H)),
            full((H, H)),
            full((1, H)),
            full((1, H)),
            full((1, H)),
        ],
        out_specs=[
            pl.BlockSpec((blk, H), lambda i: (i, 0)),
            pl.BlockSpec((blk * 4,), lambda i: (i,)),
        ],
        out_shape=[
            jax.ShapeDtypeStruct((NPAD, H), jnp.float32),
            jax.ShapeDtypeStruct((NPAD * 4,), jnp.float32),
        ],
    )(hp, pf0, pf1, p40, p41, xp4, wn1h, wn1a, bn1, wn2, bn2, gamma, beta)


def kernel(x, h, edge_index, edge_attr, Wm1, bm1, Wm2, bm2, Wn1, bn1,
           Wn2, bn2, Wc1, bc1, Wc2, bc2, gamma, beta):
    src = edge_index[0].astype(jnp.int32)
    dst = edge_index[1].astype(jnp.int32)
    x4 = jnp.zeros((NPAD, 4), jnp.float32).at[:N, :3].set(x)
    x4flat = x4.reshape(NPAD * 4)
    hp = jnp.zeros((NPAD, H), jnp.float32).at[:N].set(h)

    w_hs = Wm1[:H]
    w_hd = Wm1[H:2 * H]
    w_ea = Wm1[2 * H:2 * H + ED]
    w_d2 = Wm1[2 * H + ED:]

    tsrc, tdst = _node_proj(hp, w_hs, w_hd, bm1.reshape(1, H))
    gs, gd, d2, r0, r1, r2 = _gather(tsrc, tdst, src, dst, x4flat)
    msg, w2d = _edge_mlp(gs, gd, edge_attr, d2, w_ea, w_d2, Wm2,
                         bm2.reshape(1, H), Wc1, bc1.reshape(1, H),
                         Wc2.reshape(H, 1), bc2.reshape(1, 1))
    pf, p4 = _scatter(msg, w2d.reshape(E), r0, r1, r2, dst,
                      jnp.zeros((NPAD, H), jnp.float32),
                      jnp.zeros((NPAD * 4,), jnp.float32))
    hn, xn = _node_update(hp, pf[0], pf[1], p4[0], p4[1], x4flat,
                          Wn1[:H], Wn1[H:], bn1.reshape(1, H),
                          Wn2, bn2.reshape(1, H),
                          gamma.reshape(1, H), beta.reshape(1, H))
    return (xn.reshape(NPAD, 4)[:N, :3], hn[:N])


# speedup vs baseline: 5.1771x; 1.3678x over previous
import jax
import jax.numpy as jnp
from jax import lax
from jax.experimental import pallas as pl
from jax.experimental.pallas import tpu as pltpu
from jax.experimental.pallas import tpu_sc as plsc

N = 10000
E = 320000
H = 128
ED = 16

NPAD = 10240
NC = 2
NS = 16
NW = NC * NS
EPW = E // NW
CH = 80
NCHUNK = EPW // CH
RPT = NPAD // NS


def _silu(v):
    return v * jax.nn.sigmoid(v)


def _node_proj_body(hp, w_hs, w_hd, bm1, ts, td):
    ts[...] = jnp.dot(hp[...], w_hs[...], preferred_element_type=jnp.float32)
    td[...] = jnp.dot(hp[...], w_hd[...], preferred_element_type=jnp.float32) + bm1[...]


def _node_proj(hp, w_hs, w_hd, bm1):
    blk = 512
    return pl.pallas_call(
        _node_proj_body,
        grid=(NPAD // blk,),
        in_specs=[
            pl.BlockSpec((blk, H), lambda i: (i, 0)),
            pl.BlockSpec((H, H), lambda i: (0, 0)),
            pl.BlockSpec((H, H), lambda i: (0, 0)),
            pl.BlockSpec((1, H), lambda i: (0, 0)),
        ],
        out_specs=[
            pl.BlockSpec((blk, H), lambda i: (i, 0)),
            pl.BlockSpec((blk, H), lambda i: (i, 0)),
        ],
        out_shape=[
            jax.ShapeDtypeStruct((NPAD, H), jnp.float32),
            jax.ShapeDtypeStruct((NPAD, H), jnp.float32),
        ],
    )(hp, w_hs, w_hd, bm1)


def _gather_body(tsrc, tdst, src_h, dst_h, x4_h, gs_out, gd_out, d2_out,
                 r0_out, r1_out, r2_out,
                 idx_s, idx_d, x4, rows_s, rows_d, d2buf, rbuf, sem_s, sem_d):
    cid = lax.axis_index("c")
    sid = lax.axis_index("s")
    wid = sid * NC + cid
    base = pl.multiple_of(wid * EPW, 8)
    pltpu.sync_copy(x4_h, x4)
    pltpu.sync_copy(src_h.at[pl.ds(base, EPW)], idx_s)
    pltpu.sync_copy(dst_h.at[pl.ds(base, EPW)], idx_d)

    def chunk(j, carry):
        off = pl.multiple_of(j * CH, 8)
        cs = pltpu.async_copy(tsrc.at[idx_s.at[pl.ds(off, CH)]], rows_s, sem_s)
        cd = pltpu.async_copy(tdst.at[idx_d.at[pl.ds(off, CH)]], rows_d, sem_d)

        def grp(g, c2):
            goff = pl.multiple_of(off + g * 16, 16)
            sv = idx_s[pl.ds(goff, 16)]
            dv = idx_d[pl.ds(goff, 16)]
            acc = jnp.zeros((16,), jnp.float32)
            for comp in range(3):
                xs = plsc.load_gather(x4, [sv * 4 + comp])
                xd = plsc.load_gather(x4, [dv * 4 + comp])
                d = xs - xd
                rbuf[comp, pl.ds(pl.multiple_of(g * 16, 16), 16)] = d
                acc = acc + d * d
            d2buf[pl.ds(pl.multiple_of(g * 16, 16), 16)] = acc
            return c2

        lax.fori_loop(0, CH // 16, grp, 0)
        cs.wait()
        cd.wait()
        pltpu.sync_copy(rows_s, gs_out.at[pl.ds(base + off, CH)])
        pltpu.sync_copy(rows_d, gd_out.at[pl.ds(base + off, CH)])
        pltpu.sync_copy(d2buf, d2_out.at[pl.ds(base + off, CH)])
        pltpu.sync_copy(rbuf.at[0], r0_out.at[pl.ds(base + off, CH)])
        pltpu.sync_copy(rbuf.at[1], r1_out.at[pl.ds(base + off, CH)])
        pltpu.sync_copy(rbuf.at[2], r2_out.at[pl.ds(base + off, CH)])
        return carry

    lax.fori_loop(0, NCHUNK, chunk, 0)


def _gather(tsrc, tdst, src_i, dst_i, x4flat):
    mesh = plsc.VectorSubcoreMesh(core_axis_name="c", subcore_axis_name="s")
    f = pl.kernel(
        _gather_body,
        out_type=[
            jax.ShapeDtypeStruct((E, H), jnp.float32),
            jax.ShapeDtypeStruct((E, H), jnp.float32),
            jax.ShapeDtypeStruct((E,), jnp.float32),
            jax.ShapeDtypeStruct((E,), jnp.float32),
            jax.ShapeDtypeStruct((E,), jnp.float32),
            jax.ShapeDtypeStruct((E,), jnp.float32),
        ],
        mesh=mesh,
        compiler_params=pltpu.CompilerParams(needs_layout_passes=False),
        scratch_types=[
            pltpu.VMEM((EPW,), jnp.int32),
            pltpu.VMEM((EPW,), jnp.int32),
            pltpu.VMEM((NPAD * 4,), jnp.float32),
            pltpu.VMEM((CH, H), jnp.float32),
            pltpu.VMEM((CH, H), jnp.float32),
            pltpu.VMEM((CH,), jnp.float32),
            pltpu.VMEM((3, CH), jnp.float32),
            pltpu.SemaphoreType.DMA,
            pltpu.SemaphoreType.DMA,
        ],
    )
    return f(tsrc, tdst, src_i, dst_i, x4flat)


def _edge_body(gs, gd, ea, d2r, w_ea, w_d2, wm2, bm2, wc1, bc1, wc2, bc2,
               msg_out, w_out):
    d2 = jnp.transpose(d2r[0])
    pre = gs[...] + gd[...] \
        + jnp.dot(ea[...], w_ea[...], preferred_element_type=jnp.float32) \
        + d2 * w_d2[...]
    a1 = _silu(pre).astype(jnp.bfloat16)
    msg = jnp.dot(a1, wm2[...].astype(jnp.bfloat16),
                  preferred_element_type=jnp.float32) + bm2[...]
    c1 = _silu(jnp.dot(msg.astype(jnp.bfloat16), wc1[...].astype(jnp.bfloat16),
                       preferred_element_type=jnp.float32) + bc1[...])
    w = jnp.dot(c1.astype(jnp.bfloat16), wc2[...].astype(jnp.bfloat16),
                preferred_element_type=jnp.float32) + bc2[...]
    w = jnp.clip(w, -1.0, 1.0)
    msg_out[...] = msg
    w_out[0] = jnp.transpose(w)


def _edge_mlp(gs, gd, ea, d2, w_ea, w_d2, wm2, bm2, wc1, bc1, wc2, bc2):
    blk = 2560
    grid = E // blk
    full = lambda shape: pl.BlockSpec(shape, lambda i: tuple(0 for _ in shape))
    return pl.pallas_call(
        _edge_body,
        grid=(grid,),
        in_specs=[
            pl.BlockSpec((blk, H), lambda i: (i, 0)),
            pl.BlockSpec((blk, H), lambda i: (i, 0)),
            pl.BlockSpec((blk, ED), lambda i: (i, 0)),
            pl.BlockSpec((1, 1, blk), lambda i: (i, 0, 0)),
            full((ED, H)),
            full((1, H)),
            full((H, H)),
            full((1, H)),
            full((H, H)),
            full((1, H)),
            full((H, 1)),
            full((1, 1)),
        ],
        out_specs=[
            pl.BlockSpec((blk, H), lambda i: (i, 0)),
            pl.BlockSpec((1, 1, blk), lambda i: (i, 0, 0)),
        ],
        out_shape=[
            jax.ShapeDtypeStruct((E, H), jnp.float32),
            jax.ShapeDtypeStruct((grid, 1, blk), jnp.float32),
        ],
    )(gs, gd, ea, d2.reshape(grid, 1, blk), w_ea, w_d2, wm2, bm2, wc1, bc1,
      wc2, bc2)


def _scatter_body(msg_h, w_h, r0_h, r1_h, r2_h, dst_h, zf_h, z4_h,
                  outf_h, out4_h,
                  accf, acc4, dstc, wbuf, rc, mbuf, updc, idxc, obuf, o4buf):
    cid = lax.axis_index("c")
    sid = lax.axis_index("s")
    wid = sid * NC + cid
    base = pl.multiple_of(wid * EPW, 8)

    zrow = pl.multiple_of(sid * RPT, 8)
    pltpu.sync_copy(zf_h.at[pl.ds(zrow, RPT)], accf.at[pl.ds(zrow, RPT)])
    z4 = pl.multiple_of(sid * (NPAD * 4 // NS), 8)
    pltpu.sync_copy(z4_h.at[pl.ds(z4, NPAD * 4 // NS)],
                    acc4.at[pl.ds(z4, NPAD * 4 // NS)])
    plsc.subcore_barrier()

    def chunk(j, carry):
        off = pl.multiple_of(j * CH, 8)
        pltpu.sync_copy(dst_h.at[pl.ds(base + off, CH)], dstc)
        pltpu.sync_copy(w_h.at[pl.ds(base + off, CH)], wbuf)
        pltpu.sync_copy(r0_h.at[pl.ds(base + off, CH)], rc.at[0])
        pltpu.sync_copy(r1_h.at[pl.ds(base + off, CH)], rc.at[1])
        pltpu.sync_copy(r2_h.at[pl.ds(base + off, CH)], rc.at[2])
        pltpu.sync_copy(msg_h.at[pl.ds(base + off, CH)], mbuf)

        def grp(g, c2):
            goff = pl.multiple_of(g * 16, 16)
            dv = dstc[pl.ds(goff, 16)]
            wv = wbuf[pl.ds(goff, 16)]
            for comp in range(3):
                rv = rc[comp, pl.ds(goff, 16)]
                updc[comp, pl.ds(goff, 16)] = wv * rv
                idxc[comp, pl.ds(goff, 16)] = dv * 4 + comp
            return c2

        lax.fori_loop(0, CH // 16, grp, 0)
        pltpu.sync_copy(mbuf, accf.at[dstc], add=True)
        for comp in range(3):
            pltpu.sync_copy(updc.at[comp], acc4.at[idxc.at[comp]], add=True)
        return carry

    lax.fori_loop(0, NCHUNK, chunk, 0)
    plsc.subcore_barrier()

    for k in range(8):
        row0 = pl.multiple_of(sid * RPT + k * (RPT // 8), 8)
        pltpu.sync_copy(accf.at[pl.ds(row0, RPT // 8)], obuf)
        pltpu.sync_copy(obuf, outf_h.at[cid].at[pl.ds(row0, RPT // 8)])
    pltpu.sync_copy(acc4.at[pl.ds(z4, NPAD * 4 // NS)], o4buf)
    pltpu.sync_copy(o4buf, out4_h.at[cid].at[pl.ds(z4, NPAD * 4 // NS)])


def _scatter(msg, w1d, r0, r1, r2, dst_i, zerosf, zeros4):
    mesh = plsc.VectorSubcoreMesh(core_axis_name="c", subcore_axis_name="s")
    f = pl.kernel(
        _scatter_body,
        out_type=[
            jax.ShapeDtypeStruct((NC, NPAD, H), jnp.float32),
            jax.ShapeDtypeStruct((NC, NPAD * 4), jnp.float32),
        ],
        mesh=mesh,
        compiler_params=pltpu.CompilerParams(needs_layout_passes=False),
        scratch_types=[
            pltpu.VMEM_SHARED((NPAD, H), jnp.float32),
            pltpu.VMEM_SHARED((NPAD * 4,), jnp.float32),
            pltpu.VMEM((CH,), jnp.int32),
            pltpu.VMEM((CH,), jnp.float32),
            pltpu.VMEM((3, CH), jnp.float32),
            pltpu.VMEM((CH, H), jnp.float32),
            pltpu.VMEM((3, CH), jnp.float32),
            pltpu.VMEM((3, CH), jnp.int32),
            pltpu.VMEM((RPT // 8, H), jnp.float32),
            pltpu.VMEM((NPAD * 4 // NS,), jnp.float32),
        ],
    )
    return f(msg, w1d, r0, r1, r2, dst_i, zerosf, zeros4)


def _update_body(hp, pf0, pf1, p40, p41, xp4, wn1h, wn1a, bn1, wn2, bn2,
                 gamma, beta, hout, xout):
    agg = pf0[...] + pf1[...]
    u = _silu(jnp.dot(hp[...], wn1h[...], preferred_element_type=jnp.float32)
              + jnp.dot(agg, wn1a[...], preferred_element_type=jnp.float32)
              + bn1[...])
    hn = hp[...] + jnp.dot(u, wn2[...], preferred_element_type=jnp.float32) + bn2[...]
    mean = jnp.mean(hn, axis=1, keepdims=True)
    zc = hn - mean
    var = jnp.mean(zc * zc, axis=1, keepdims=True)
    hout[...] = zc * lax.rsqrt(var + 1e-5) * gamma[...] + beta[...]
    xout[...] = xp4[...] + p40[...] + p41[...]


def _node_update(hp, pf0, pf1, p40, p41, xp4, wn1h, wn1a, bn1, wn2, bn2,
                 gamma, beta):
    blk = 512
    grid = NPAD // blk
    full = lambda shape: pl.BlockSpec(shape, lambda i: tuple(0 for _ in shape))
    return pl.pallas_call(
        _update_body,
        grid=(grid,),
        in_specs=[
            pl.BlockSpec((blk, H), lambda i: (i, 0)),
            pl.BlockSpec((blk, H), lambda i: (i, 0)),
            pl.BlockSpec((blk, H), lambda i: (i, 0)),
            pl.BlockSpec((blk * 4,), lambda i: (i,)),
            pl.BlockSpec((blk * 4,), lambda i: (i,)),
            pl.BlockSpec((blk * 4,), lambda i: (i,)),
            full((H, H)),
            full((H, H)),
            full((1, H)),
            full((H, H)),
            full((1, H)),
            full((1, H)),
            full((1, H)),
        ],
        out_specs=[
            pl.BlockSpec((blk, H), lambda i: (i, 0)),
            pl.BlockSpec((blk * 4,), lambda i: (i,)),
        ],
        out_shape=[
            jax.ShapeDtypeStruct((NPAD, H), jnp.float32),
            jax.ShapeDtypeStruct((NPAD * 4,), jnp.float32),
        ],
    )(hp, pf0, pf1, p40, p41, xp4, wn1h, wn1a, bn1, wn2, bn2, gamma, beta)


def kernel(x, h, edge_index, edge_attr, Wm1, bm1, Wm2, bm2, Wn1, bn1,
           Wn2, bn2, Wc1, bc1, Wc2, bc2, gamma, beta):
    src = edge_index[0].astype(jnp.int32)
    dst = edge_index[1].astype(jnp.int32)
    x4 = jnp.zeros((NPAD, 4), jnp.float32).at[:N, :3].set(x)
    x4flat = x4.reshape(NPAD * 4)
    hp = jnp.zeros((NPAD, H), jnp.float32).at[:N].set(h)

    w_hs = Wm1[:H]
    w_hd = Wm1[H:2 * H]
    w_ea = Wm1[2 * H:2 * H + ED]
    w_d2 = Wm1[2 * H + ED:]

    tsrc, tdst = _node_proj(hp, w_hs, w_hd, bm1.reshape(1, H))
    gs, gd, d2, r0, r1, r2 = _gather(tsrc, tdst, src, dst, x4flat)
    msg, w2d = _edge_mlp(gs, gd, edge_attr, d2, w_ea, w_d2, Wm2,
                         bm2.reshape(1, H), Wc1, bc1.reshape(1, H),
                         Wc2.reshape(H, 1), bc2.reshape(1, 1))
    pf, p4 = _scatter(msg, w2d.reshape(E), r0, r1, r2, dst,
                      jnp.zeros((NPAD, H), jnp.float32),
                      jnp.zeros((NPAD * 4,), jnp.float32))
    hn, xn = _node_update(hp, pf[0], pf[1], p4[0], p4[1], x4flat,
                          Wn1[:H], Wn1[H:], bn1.reshape(1, H),
                          Wn2, bn2.reshape(1, H),
                          gamma.reshape(1, H), beta.reshape(1, H))
    return (xn.reshape(NPAD, 4)[:N, :3], hn[:N])


# speedup vs baseline: 6.0349x; 1.1657x over previous
import jax
import jax.numpy as jnp
from jax import lax
from jax.experimental import pallas as pl
from jax.experimental.pallas import tpu as pltpu
from jax.experimental.pallas import tpu_sc as plsc

N = 10000
E = 320000
H = 128
ED = 16

NPAD = 10240
NC = 2
NS = 16
NW = NC * NS
EPW = E // NW
CH = 80
NCHUNK = EPW // CH
RPT = NPAD // NS


def _silu(v):
    return v * jax.nn.sigmoid(v)


def _node_proj_body(hp, w_hs, w_hd, bm1, ts, td):
    ts[...] = jnp.dot(hp[...], w_hs[...], preferred_element_type=jnp.float32)
    td[...] = jnp.dot(hp[...], w_hd[...], preferred_element_type=jnp.float32) + bm1[...]


def _node_proj(hp, w_hs, w_hd, bm1):
    blk = 512
    return pl.pallas_call(
        _node_proj_body,
        grid=(NPAD // blk,),
        in_specs=[
            pl.BlockSpec((blk, H), lambda i: (i, 0)),
            pl.BlockSpec((H, H), lambda i: (0, 0)),
            pl.BlockSpec((H, H), lambda i: (0, 0)),
            pl.BlockSpec((1, H), lambda i: (0, 0)),
        ],
        out_specs=[
            pl.BlockSpec((blk, H), lambda i: (i, 0)),
            pl.BlockSpec((blk, H), lambda i: (i, 0)),
        ],
        out_shape=[
            jax.ShapeDtypeStruct((NPAD, H), jnp.float32),
            jax.ShapeDtypeStruct((NPAD, H), jnp.float32),
        ],
    )(hp, w_hs, w_hd, bm1)


def _gather_body(tsrc, tdst, src_h, dst_h, x4_h, gs_out, gd_out, d2_out,
                 r0_out, r1_out, r2_out,
                 idx_s, idx_d, x4,
                 rows_s0, rows_d0, dr0, rows_s1, rows_d1, dr1,
                 gsem0, gsem1, wsem0, wsem1):
    cid = lax.axis_index("c")
    sid = lax.axis_index("s")
    wid = sid * NC + cid
    base = pl.multiple_of(wid * EPW, 8)
    pltpu.sync_copy(x4_h, x4)
    pltpu.sync_copy(src_h.at[pl.ds(base, EPW)], idx_s)
    pltpu.sync_copy(dst_h.at[pl.ds(base, EPW)], idx_d)

    slot_a = (rows_s0, rows_d0, dr0, gsem0, wsem0)
    slot_b = (rows_s1, rows_d1, dr1, gsem1, wsem1)
    outs = (d2_out, r0_out, r1_out, r2_out)

    def issue_gathers(slot, c):
        rows_s, rows_d, _, gsem, _ = slot
        off = pl.multiple_of(c * CH, 8)
        pltpu.async_copy(tsrc.at[idx_s.at[pl.ds(off, CH)]], rows_s, gsem)
        pltpu.async_copy(tdst.at[idx_d.at[pl.ds(off, CH)]], rows_d, gsem)

    def wait_gathers(slot):
        rows_s, rows_d, _, gsem, _ = slot
        pltpu.make_async_copy(tsrc.at[idx_s.at[pl.ds(0, CH)]], rows_s,
                              gsem).wait()
        pltpu.make_async_copy(tdst.at[idx_d.at[pl.ds(0, CH)]], rows_d,
                              gsem).wait()

    def compute_dr(slot, c):
        _, _, dr, _, _ = slot
        off = pl.multiple_of(c * CH, 8)

        def grp(g, c2):
            goff = pl.multiple_of(off + g * 16, 16)
            lo = pl.ds(pl.multiple_of(g * 16, 16), 16)
            sv = idx_s[pl.ds(goff, 16)]
            dv = idx_d[pl.ds(goff, 16)]
            acc = jnp.zeros((16,), jnp.float32)
            for comp in range(3):
                xs = plsc.load_gather(x4, [sv * 4 + comp])
                xd = plsc.load_gather(x4, [dv * 4 + comp])
                d = xs - xd
                dr[1 + comp, lo] = d
                acc = acc + d * d
            dr[0, lo] = acc
            return c2

        lax.fori_loop(0, CH // 16, grp, 0)

    def issue_writes(slot, c):
        rows_s, rows_d, dr, _, wsem = slot
        off = pl.multiple_of(base + c * CH, 8)
        pltpu.async_copy(rows_s, gs_out.at[pl.ds(off, CH)], wsem)
        pltpu.async_copy(rows_d, gd_out.at[pl.ds(off, CH)], wsem)
        for k in range(4):
            pltpu.async_copy(dr.at[k], outs[k].at[pl.ds(off, CH)], wsem)

    def wait_writes(slot):
        rows_s, rows_d, dr, _, wsem = slot
        pltpu.make_async_copy(rows_s, gs_out.at[pl.ds(0, CH)], wsem).wait()
        pltpu.make_async_copy(rows_d, gd_out.at[pl.ds(0, CH)], wsem).wait()
        for k in range(4):
            pltpu.make_async_copy(dr.at[k], outs[k].at[pl.ds(0, CH)],
                                  wsem).wait()

    issue_gathers(slot_a, 0)

    def pair(k, carry):
        @pl.when(k > 0)
        def _():
            wait_writes(slot_b)

        issue_gathers(slot_b, 2 * k + 1)
        compute_dr(slot_a, 2 * k)
        wait_gathers(slot_a)
        issue_writes(slot_a, 2 * k)
        wait_writes(slot_a)
        issue_gathers(slot_a, 2 * k + 2)
        compute_dr(slot_b, 2 * k + 1)
        wait_gathers(slot_b)
        issue_writes(slot_b, 2 * k + 1)
        return carry

    lax.fori_loop(0, (NCHUNK - 1) // 2, pair, 0)
    wait_writes(slot_b)
    compute_dr(slot_a, NCHUNK - 1)
    wait_gathers(slot_a)
    issue_writes(slot_a, NCHUNK - 1)
    wait_writes(slot_a)


def _gather(tsrc, tdst, src_i, dst_i, x4flat):
    mesh = plsc.VectorSubcoreMesh(core_axis_name="c", subcore_axis_name="s")
    f = pl.kernel(
        _gather_body,
        out_type=[
            jax.ShapeDtypeStruct((E, H), jnp.float32),
            jax.ShapeDtypeStruct((E, H), jnp.float32),
            jax.ShapeDtypeStruct((E,), jnp.float32),
            jax.ShapeDtypeStruct((E,), jnp.float32),
            jax.ShapeDtypeStruct((E,), jnp.float32),
            jax.ShapeDtypeStruct((E,), jnp.float32),
        ],
        mesh=mesh,
        compiler_params=pltpu.CompilerParams(needs_layout_passes=False),
        scratch_types=[
            pltpu.VMEM((EPW,), jnp.int32),
            pltpu.VMEM((EPW,), jnp.int32),
            pltpu.VMEM((NPAD * 4,), jnp.float32),
            pltpu.VMEM((CH, H), jnp.float32),
            pltpu.VMEM((CH, H), jnp.float32),
            pltpu.VMEM((4, CH), jnp.float32),
            pltpu.VMEM((CH, H), jnp.float32),
            pltpu.VMEM((CH, H), jnp.float32),
            pltpu.VMEM((4, CH), jnp.float32),
            pltpu.SemaphoreType.DMA,
            pltpu.SemaphoreType.DMA,
            pltpu.SemaphoreType.DMA,
            pltpu.SemaphoreType.DMA,
        ],
    )
    return f(tsrc, tdst, src_i, dst_i, x4flat)


def _edge_body(gs, gd, ea, d2r, r0r, r1r, r2r, w_ea, w_d2, wm2, bm2, wc1,
               bc1, wc2, bc2, msg_out, wr0_out, wr1_out, wr2_out):
    d2 = jnp.transpose(d2r[0])
    pre = gs[...] + gd[...] \
        + jnp.dot(ea[...], w_ea[...], preferred_element_type=jnp.float32) \
        + d2 * w_d2[...]
    a1 = _silu(pre).astype(jnp.bfloat16)
    msg = jnp.dot(a1, wm2[...].astype(jnp.bfloat16),
                  preferred_element_type=jnp.float32) + bm2[...]
    c1 = _silu(jnp.dot(msg.astype(jnp.bfloat16), wc1[...].astype(jnp.bfloat16),
                       preferred_element_type=jnp.float32) + bc1[...])
    w = jnp.dot(c1.astype(jnp.bfloat16), wc2[...].astype(jnp.bfloat16),
                preferred_element_type=jnp.float32) + bc2[...]
    w = jnp.clip(w, -1.0, 1.0)
    msg_out[...] = msg
    wrow = jnp.transpose(w)
    wr0_out[0] = wrow * r0r[0]
    wr1_out[0] = wrow * r1r[0]
    wr2_out[0] = wrow * r2r[0]


def _edge_mlp(gs, gd, ea, d2, r0, r1, r2, w_ea, w_d2, wm2, bm2, wc1, bc1,
              wc2, bc2):
    blk = 2560
    grid = E // blk
    full = lambda shape: pl.BlockSpec(shape, lambda i: tuple(0 for _ in shape))
    sc1 = pl.BlockSpec((1, 1, blk), lambda i: (i, 0, 0))
    scalar_out = jax.ShapeDtypeStruct((grid, 1, blk), jnp.float32)
    return pl.pallas_call(
        _edge_body,
        grid=(grid,),
        in_specs=[
            pl.BlockSpec((blk, H), lambda i: (i, 0)),
            pl.BlockSpec((blk, H), lambda i: (i, 0)),
            pl.BlockSpec((blk, ED), lambda i: (i, 0)),
            sc1, sc1, sc1, sc1,
            full((ED, H)),
            full((1, H)),
            full((H, H)),
            full((1, H)),
            full((H, H)),
            full((1, H)),
            full((H, 1)),
            full((1, 1)),
        ],
        out_specs=[
            pl.BlockSpec((blk, H), lambda i: (i, 0)),
            sc1, sc1, sc1,
        ],
        out_shape=[
            jax.ShapeDtypeStruct((E, H), jnp.float32),
            scalar_out, scalar_out, scalar_out,
        ],
    )(gs, gd, ea, d2.reshape(grid, 1, blk), r0.reshape(grid, 1, blk),
      r1.reshape(grid, 1, blk), r2.reshape(grid, 1, blk), w_ea, w_d2,
      wm2, bm2, wc1, bc1, wc2, bc2)


def _scatter_body(msg_h, wr0_h, wr1_h, wr2_h, dst_h, zf_h, z4_h,
                  outf_h, out4_h,
                  accf, acc4,
                  dstc0, mbuf0, updc0, idxc0,
                  dstc1, mbuf1, updc1, idxc1,
                  obuf, o4buf, lsem0, lsem1, ssem0, ssem1):
    cid = lax.axis_index("c")
    sid = lax.axis_index("s")
    wid = sid * NC + cid
    base = pl.multiple_of(wid * EPW, 8)

    zrow = pl.multiple_of(sid * RPT, 8)
    pltpu.sync_copy(zf_h.at[pl.ds(zrow, RPT)], accf.at[pl.ds(zrow, RPT)])
    z4 = pl.multiple_of(sid * (NPAD * 4 // NS), 8)
    pltpu.sync_copy(z4_h.at[pl.ds(z4, NPAD * 4 // NS)],
                    acc4.at[pl.ds(z4, NPAD * 4 // NS)])
    plsc.subcore_barrier()

    slot_a = (dstc0, mbuf0, updc0, idxc0, lsem0, ssem0)
    slot_b = (dstc1, mbuf1, updc1, idxc1, lsem1, ssem1)

    def issue_loads(slot, c):
        dstc, mbuf, updc, _, lsem, _ = slot
        off = pl.multiple_of(base + c * CH, 8)
        pltpu.async_copy(dst_h.at[pl.ds(off, CH)], dstc, lsem)
        pltpu.async_copy(msg_h.at[pl.ds(off, CH)], mbuf, lsem)
        pltpu.async_copy(wr0_h.at[pl.ds(off, CH)], updc.at[0], lsem)
        pltpu.async_copy(wr1_h.at[pl.ds(off, CH)], updc.at[1], lsem)
        pltpu.async_copy(wr2_h.at[pl.ds(off, CH)], updc.at[2], lsem)

    def wait_loads(slot):
        dstc, mbuf, updc, _, lsem, _ = slot
        pltpu.make_async_copy(dst_h.at[pl.ds(0, CH)], dstc, lsem).wait()
        pltpu.make_async_copy(msg_h.at[pl.ds(0, CH)], mbuf, lsem).wait()
        for k in range(3):
            pltpu.make_async_copy(wr0_h.at[pl.ds(0, CH)], updc.at[k],
                                  lsem).wait()

    def do_scatter(slot):
        dstc, mbuf, updc, idxc, _, ssem = slot

        def grp(g, c2):
            goff = pl.multiple_of(g * 16, 16)
            dv = dstc[pl.ds(goff, 16)]
            for comp in range(3):
                idxc[comp, pl.ds(goff, 16)] = dv * 4 + comp
            return c2

        lax.fori_loop(0, CH // 16, grp, 0)
        pltpu.async_copy(mbuf, accf.at[dstc], ssem, add=True)
        for comp in range(3):
            pltpu.async_copy(updc.at[comp], acc4.at[idxc.at[comp]], ssem,
                             add=True)

    def wait_scatter(slot):
        dstc, mbuf, updc, idxc, _, ssem = slot
        pltpu.make_async_copy(mbuf, accf.at[dstc], ssem).wait()
        for comp in range(3):
            pltpu.make_async_copy(updc.at[comp], acc4.at[idxc.at[comp]],
                                  ssem).wait()

    issue_loads(slot_a, 0)

    def pair(k, carry):
        @pl.when(k > 0)
        def _():
            wait_scatter(slot_b)

        issue_loads(slot_b, 2 * k + 1)
        wait_loads(slot_a)
        do_scatter(slot_a)
        wait_scatter(slot_a)
        issue_loads(slot_a, 2 * k + 2)
        wait_loads(slot_b)
        do_scatter(slot_b)
        return carry

    lax.fori_loop(0, (NCHUNK - 1) // 2, pair, 0)
    wait_scatter(slot_b)
    wait_loads(slot_a)
    do_scatter(slot_a)
    wait_scatter(slot_a)
    plsc.subcore_barrier()

    for k in range(8):
        row0 = pl.multiple_of(sid * RPT + k * (RPT // 8), 8)
        pltpu.sync_copy(accf.at[pl.ds(row0, RPT // 8)], obuf)
        pltpu.sync_copy(obuf, outf_h.at[cid].at[pl.ds(row0, RPT // 8)])
    pltpu.sync_copy(acc4.at[pl.ds(z4, NPAD * 4 // NS)], o4buf)
    pltpu.sync_copy(o4buf, out4_h.at[cid].at[pl.ds(z4, NPAD * 4 // NS)])


def _scatter(msg, wr0, wr1, wr2, dst_i, zerosf, zeros4):
    mesh = plsc.VectorSubcoreMesh(core_axis_name="c", subcore_axis_name="s")
    slot = [
        pltpu.VMEM((CH,), jnp.int32),
        pltpu.VMEM((CH, H), jnp.float32),
        pltpu.VMEM((3, CH), jnp.float32),
        pltpu.VMEM((3, CH), jnp.int32),
    ]
    f = pl.kernel(
        _scatter_body,
        out_type=[
            jax.ShapeDtypeStruct((NC, NPAD, H), jnp.float32),
            jax.ShapeDtypeStruct((NC, NPAD * 4), jnp.float32),
        ],
        mesh=mesh,
        compiler_params=pltpu.CompilerParams(needs_layout_passes=False),
        scratch_types=[
            pltpu.VMEM_SHARED((NPAD, H), jnp.float32),
            pltpu.VMEM_SHARED((NPAD * 4,), jnp.float32),
        ] + slot + slot + [
            pltpu.VMEM((RPT // 8, H), jnp.float32),
            pltpu.VMEM((NPAD * 4 // NS,), jnp.float32),
            pltpu.SemaphoreType.DMA,
            pltpu.SemaphoreType.DMA,
            pltpu.SemaphoreType.DMA,
            pltpu.SemaphoreType.DMA,
        ],
    )
    return f(msg, wr0, wr1, wr2, dst_i, zerosf, zeros4)


def _update_body(hp, pf0, pf1, p40, p41, xp4, wn1h, wn1a, bn1, wn2, bn2,
                 gamma, beta, hout, xout):
    agg = pf0[...] + pf1[...]
    u = _silu(jnp.dot(hp[...], wn1h[...], preferred_element_type=jnp.float32)
              + jnp.dot(agg, wn1a[...], preferred_element_type=jnp.float32)
              + bn1[...])
    hn = hp[...] + jnp.dot(u, wn2[...], preferred_element_type=jnp.float32) + bn2[...]
    mean = jnp.mean(hn, axis=1, keepdims=True)
    zc = hn - mean
    var = jnp.mean(zc * zc, axis=1, keepdims=True)
    hout[...] = zc * lax.rsqrt(var + 1e-5) * gamma[...] + beta[...]
    xout[...] = xp4[...] + p40[...] + p41[...]


def _node_update(hp, pf0, pf1, p40, p41, xp4, wn1h, wn1a, bn1, wn2, bn2,
                 gamma, beta):
    blk = 512
    grid = NPAD // blk
    full = lambda shape: pl.BlockSpec(shape, lambda i: tuple(0 for _ in shape))
    return pl.pallas_call(
        _update_body,
        grid=(grid,),
        in_specs=[
            pl.BlockSpec((blk, H), lambda i: (i, 0)),
            pl.BlockSpec((blk, H), lambda i: (i, 0)),
            pl.BlockSpec((blk, H), lambda i: (i, 0)),
            pl.BlockSpec((blk * 4,), lambda i: (i,)),
            pl.BlockSpec((blk * 4,), lambda i: (i,)),
            pl.BlockSpec((blk * 4,), lambda i: (i,)),
            full((H, H)),
            full((H, H)),
            full((1, H)),
            full((H, H)),
            full((1, H)),
            full((1, H)),
            full((1, H)),
        ],
        out_specs=[
            pl.BlockSpec((blk, H), lambda i: (i, 0)),
            pl.BlockSpec((blk * 4,), lambda i: (i,)),
        ],
        out_shape=[
            jax.ShapeDtypeStruct((NPAD, H), jnp.float32),
            jax.ShapeDtypeStruct((NPAD * 4,), jnp.float32),
        ],
    )(hp, pf0, pf1, p40, p41, xp4, wn1h, wn1a, bn1, wn2, bn2, gamma, beta)


def kernel(x, h, edge_index, edge_attr, Wm1, bm1, Wm2, bm2, Wn1, bn1,
           Wn2, bn2, Wc1, bc1, Wc2, bc2, gamma, beta):
    src = edge_index[0].astype(jnp.int32)
    dst = edge_index[1].astype(jnp.int32)
    x4 = jnp.zeros((NPAD, 4), jnp.float32).at[:N, :3].set(x)
    x4flat = x4.reshape(NPAD * 4)
    hp = jnp.zeros((NPAD, H), jnp.float32).at[:N].set(h)

    w_hs = Wm1[:H]
    w_hd = Wm1[H:2 * H]
    w_ea = Wm1[2 * H:2 * H + ED]
    w_d2 = Wm1[2 * H + ED:]

    tsrc, tdst = _node_proj(hp, w_hs, w_hd, bm1.reshape(1, H))
    gs, gd, d2, r0, r1, r2 = _gather(tsrc, tdst, src, dst, x4flat)
    msg, wr0, wr1, wr2 = _edge_mlp(gs, gd, edge_attr, d2, r0, r1, r2,
                                   w_ea, w_d2, Wm2, bm2.reshape(1, H),
                                   Wc1, bc1.reshape(1, H),
                                   Wc2.reshape(H, 1), bc2.reshape(1, 1))
    pf, p4 = _scatter(msg, wr0.reshape(E), wr1.reshape(E), wr2.reshape(E),
                      dst, jnp.zeros((NPAD, H), jnp.float32),
                      jnp.zeros((NPAD * 4,), jnp.float32))
    hn, xn = _node_update(hp, pf[0], pf[1], p4[0], p4[1], x4flat,
                          Wn1[:H], Wn1[H:], bn1.reshape(1, H),
                          Wn2, bn2.reshape(1, H),
                          gamma.reshape(1, H), beta.reshape(1, H))
    return (xn.reshape(NPAD, 4)[:N, :3], hn[:N])
